# baseline (speedup 1.0000x reference)
import jax
import jax.numpy as jnp
from jax.experimental import pallas as pl
from jax.experimental.pallas import tpu as pltpu

N = 1024
IN_CH = 512
OUT_CH = 512
NUM_CHANNELS = 2


def _softmax_row(row):
    m = jnp.max(row, axis=1, keepdims=True)
    e = jnp.exp(row - m)
    return e / jnp.sum(e, axis=1, keepdims=True)


def _coeff_rows(w1_row, w2_row, w3_row):
    f = _softmax_row(w1_row)
    g = _softmax_row(w2_row)
    h = _softmax_row(w3_row)
    S_g = jnp.sum(g, axis=1, keepdims=True)
    S_f = jnp.sum(f, axis=1, keepdims=True)
    u = S_g * f - f * g
    S_u = jnp.sum(u, axis=1, keepdims=True)
    d = (S_u - u) - g * (S_f - f)
    inv_d = jnp.where(d == 0.0, 0.0, 1.0 / d)
    r = h * inv_d
    w = g * r
    R_ = jnp.sum(r, axis=1, keepdims=True)
    Wt = jnp.sum(w, axis=1, keepdims=True)
    v = u * (R_ - r) - f * (Wt - w)
    S_v = jnp.sum(v, axis=1, keepdims=True)
    deg2 = 1.0 + (S_v - v) - r * (S_u - u) + w * (S_f - f)
    inv2 = jnp.where(deg2 == 0.0, 0.0, 1.0 / deg2)
    beta = inv2 * (1.0 - v + r * u - w * f)
    zero = jnp.zeros_like(f)
    return jnp.concatenate(
        [inv2, inv2 * r, inv2 * w, beta, v, u, f, zero], axis=0)


def _tdot(a, b):
    return jax.lax.dot_general(a, b, (((0,), (0,)), ((), ())),
                               preferred_element_type=jnp.float32)


NCHUNK = 1
CHUNK = N // NCHUNK


def _body(x_hbm, w1_ref, w2_ref, w3_ref, wgcn_hbm, linw_hbm, linb_ref,
          out_hbm, xv, gv, lv, ov, sx, sg, sl, so):
    cp_g = pltpu.make_async_copy(wgcn_hbm, gv, sg)
    cp_x = pltpu.make_async_copy(x_hbm, xv, sx)
    cp_l = pltpu.make_async_copy(linw_hbm, lv, sl)
    cp_g.start()
    cp_x.start()

    trows = [
        _coeff_rows(w1_ref[c:c + 1, :], w2_ref[c:c + 1, :], w3_ref[c:c + 1, :])
        for c in range(NUM_CHANNELS)
    ]
    eye8 = jnp.eye(8, dtype=jnp.float32)
    tcols = [_tdot(t, eye8) for t in trows]

    cp_g.wait()
    g_bf = gv[...].astype(jnp.bfloat16)
    cp_x.wait()
    cp_l.start()
    Xw = jnp.dot(xv[...].astype(jnp.bfloat16), g_bf,
                 preferred_element_type=jnp.float32)

    Ms = []
    for c in range(NUM_CHANNELS):
        R8 = jnp.dot(trows[c], Xw, preferred_element_type=jnp.float32)
        Ms.append(jnp.concatenate(
            [R8[4:5, :], -R8[5:6, :], R8[6:7, :],
             jnp.zeros((5, OUT_CH), dtype=jnp.float32)], axis=0))

    outs = []
    for c in range(NUM_CHANNELS):
        corr = jnp.dot(tcols[c], Ms[c], preferred_element_type=jnp.float32)
        beta = tcols[c][:, 3:4]
        outs.append(
            jnp.maximum(beta * Xw + corr, 0.0).astype(jnp.bfloat16))

    cp_l.wait()
    l_bf = lv[...].astype(jnp.bfloat16)
    cp_o = []
    for k in range(NCHUNK):
        sl_k = pl.ds(k * CHUNK, CHUNK)
        acc = None
        for c in range(NUM_CHANNELS):
            part = jnp.dot(outs[c][k * CHUNK:(k + 1) * CHUNK, :],
                           l_bf[c * OUT_CH:(c + 1) * OUT_CH, :],
                           preferred_element_type=jnp.float32)
            acc = part if acc is None else acc + part
        ov[sl_k, :] = jnp.maximum(acc + linb_ref[...], 0.0)
        cp = pltpu.make_async_copy(ov.at[sl_k, :], out_hbm.at[sl_k, :],
                                   so.at[k])
        cp.start()
        cp_o.append(cp)
    for cp in cp_o:
        cp.wait()


def kernel(x, W1, W2, W3, Wgcn, lin_w, lin_b):
    return pl.pallas_call(
        _body,
        in_specs=[
            pl.BlockSpec(memory_space=pltpu.HBM),
            pl.BlockSpec(memory_space=pltpu.VMEM),
            pl.BlockSpec(memory_space=pltpu.VMEM),
            pl.BlockSpec(memory_space=pltpu.VMEM),
            pl.BlockSpec(memory_space=pltpu.HBM),
            pl.BlockSpec(memory_space=pltpu.HBM),
            pl.BlockSpec(memory_space=pltpu.VMEM),
        ],
        out_specs=pl.BlockSpec(memory_space=pltpu.HBM),
        out_shape=jax.ShapeDtypeStruct((N, OUT_CH), jnp.float32),
        scratch_shapes=[
            pltpu.VMEM((N, IN_CH), jnp.float32),
            pltpu.VMEM((IN_CH, OUT_CH), jnp.float32),
            pltpu.VMEM((OUT_CH * NUM_CHANNELS, OUT_CH), jnp.float32),
            pltpu.VMEM((N, OUT_CH), jnp.float32),
            pltpu.SemaphoreType.DMA,
            pltpu.SemaphoreType.DMA,
            pltpu.SemaphoreType.DMA,
            pltpu.SemaphoreType.DMA((NCHUNK,)),
        ],
    )(x, W1, W2, W3, Wgcn, lin_w, lin_b.reshape(1, OUT_CH))


# speedup vs baseline: 1.0094x; 1.0094x over previous
import jax
import jax.numpy as jnp
from jax.experimental import pallas as pl
from jax.experimental.pallas import tpu as pltpu

N = 1024
IN_CH = 512
OUT_CH = 512
NUM_CHANNELS = 2


def _softmax_row(row):
    m = jnp.max(row, axis=1, keepdims=True)
    e = jnp.exp(row - m)
    return e / jnp.sum(e, axis=1, keepdims=True)


def _coeff_rows(w1_row, w2_row, w3_row):
    f = _softmax_row(w1_row)
    g = _softmax_row(w2_row)
    h = _softmax_row(w3_row)
    S_g = jnp.sum(g, axis=1, keepdims=True)
    S_f = jnp.sum(f, axis=1, keepdims=True)
    u = S_g * f - f * g
    S_u = jnp.sum(u, axis=1, keepdims=True)
    d = (S_u - u) - g * (S_f - f)
    inv_d = jnp.where(d == 0.0, 0.0, 1.0 / d)
    r = h * inv_d
    w = g * r
    R_ = jnp.sum(r, axis=1, keepdims=True)
    Wt = jnp.sum(w, axis=1, keepdims=True)
    v = u * (R_ - r) - f * (Wt - w)
    S_v = jnp.sum(v, axis=1, keepdims=True)
    deg2 = 1.0 + (S_v - v) - r * (S_u - u) + w * (S_f - f)
    inv2 = jnp.where(deg2 == 0.0, 0.0, 1.0 / deg2)
    beta = inv2 * (1.0 - v + r * u - w * f)
    zero = jnp.zeros_like(f)
    return jnp.concatenate(
        [inv2, inv2 * r, inv2 * w, beta, v, u, f, zero], axis=0)


def _tdot(a, b):
    return jax.lax.dot_general(a, b, (((0,), (0,)), ((), ())),
                               preferred_element_type=jnp.float32)


NCHUNK = 2
CHUNK = N // NCHUNK


def _body(x_hbm, w1_ref, w2_ref, w3_ref, wgcn_hbm, linw_hbm, linb_ref,
          out_hbm, xv, gv, lv, ov, sx, sg, sl, so):
    half = N // 2
    cp_g = pltpu.make_async_copy(wgcn_hbm, gv, sg)
    cp_x0 = pltpu.make_async_copy(x_hbm.at[pl.ds(0, half), :],
                                  xv.at[pl.ds(0, half), :], sx.at[0])
    cp_x1 = pltpu.make_async_copy(x_hbm.at[pl.ds(half, half), :],
                                  xv.at[pl.ds(half, half), :], sx.at[1])
    cp_l = pltpu.make_async_copy(linw_hbm, lv, sl)
    cp_g.start()
    cp_x0.start()
    cp_x1.start()

    trows = [
        _coeff_rows(w1_ref[c:c + 1, :], w2_ref[c:c + 1, :], w3_ref[c:c + 1, :])
        for c in range(NUM_CHANNELS)
    ]
    eye8 = jnp.eye(8, dtype=jnp.float32)
    tcols = [_tdot(t, eye8) for t in trows]

    cp_g.wait()
    g_bf = gv[...].astype(jnp.bfloat16)
    cp_x0.wait()
    xw0 = jnp.dot(xv[0:half, :].astype(jnp.bfloat16), g_bf,
                  preferred_element_type=jnp.float32)
    cp_x1.wait()
    cp_l.start()
    xw1 = jnp.dot(xv[half:N, :].astype(jnp.bfloat16), g_bf,
                  preferred_element_type=jnp.float32)
    Xw = jnp.concatenate([xw0, xw1], axis=0)

    Ms = []
    for c in range(NUM_CHANNELS):
        R8 = jnp.dot(trows[c], Xw, preferred_element_type=jnp.float32)
        Ms.append(jnp.concatenate(
            [R8[4:5, :], -R8[5:6, :], R8[6:7, :],
             jnp.zeros((5, OUT_CH), dtype=jnp.float32)], axis=0))

    outs = []
    for c in range(NUM_CHANNELS):
        corr = jnp.dot(tcols[c], Ms[c], preferred_element_type=jnp.float32)
        beta = tcols[c][:, 3:4]
        outs.append(
            jnp.maximum(beta * Xw + corr, 0.0).astype(jnp.bfloat16))

    cp_l.wait()
    l_bf = lv[...].astype(jnp.bfloat16)
    cp_o = []
    for k in range(NCHUNK):
        sl_k = pl.ds(k * CHUNK, CHUNK)
        acc = None
        for c in range(NUM_CHANNELS):
            part = jnp.dot(outs[c][k * CHUNK:(k + 1) * CHUNK, :],
                           l_bf[c * OUT_CH:(c + 1) * OUT_CH, :],
                           preferred_element_type=jnp.float32)
            acc = part if acc is None else acc + part
        ov[sl_k, :] = jnp.maximum(acc + linb_ref[...], 0.0)
        cp = pltpu.make_async_copy(ov.at[sl_k, :], out_hbm.at[sl_k, :],
                                   so.at[k])
        cp.start()
        cp_o.append(cp)
    for cp in cp_o:
        cp.wait()


def kernel(x, W1, W2, W3, Wgcn, lin_w, lin_b):
    return pl.pallas_call(
        _body,
        in_specs=[
            pl.BlockSpec(memory_space=pltpu.HBM),
            pl.BlockSpec(memory_space=pltpu.VMEM),
            pl.BlockSpec(memory_space=pltpu.VMEM),
            pl.BlockSpec(memory_space=pltpu.VMEM),
            pl.BlockSpec(memory_space=pltpu.HBM),
            pl.BlockSpec(memory_space=pltpu.HBM),
            pl.BlockSpec(memory_space=pltpu.VMEM),
        ],
        out_specs=pl.BlockSpec(memory_space=pltpu.HBM),
        out_shape=jax.ShapeDtypeStruct((N, OUT_CH), jnp.float32),
        scratch_shapes=[
            pltpu.VMEM((N, IN_CH), jnp.float32),
            pltpu.VMEM((IN_CH, OUT_CH), jnp.float32),
            pltpu.VMEM((OUT_CH * NUM_CHANNELS, OUT_CH), jnp.float32),
            pltpu.VMEM((N, OUT_CH), jnp.float32),
            pltpu.SemaphoreType.DMA((2,)),
            pltpu.SemaphoreType.DMA,
            pltpu.SemaphoreType.DMA,
            pltpu.SemaphoreType.DMA((NCHUNK,)),
        ],
    )(x, W1, W2, W3, Wgcn, lin_w, lin_b.reshape(1, OUT_CH))


# speedup vs baseline: 1.0452x; 1.0355x over previous
import jax
import jax.numpy as jnp
from jax.experimental import pallas as pl
from jax.experimental.pallas import tpu as pltpu

N = 1024
IN_CH = 512
OUT_CH = 512
NUM_CHANNELS = 2


def _softmax_row(row):
    m = jnp.max(row, axis=1, keepdims=True)
    e = jnp.exp(row - m)
    return e / jnp.sum(e, axis=1, keepdims=True)


def _coeff_rows(w1_row, w2_row, w3_row):
    f = _softmax_row(w1_row)
    g = _softmax_row(w2_row)
    h = _softmax_row(w3_row)
    S_g = jnp.sum(g, axis=1, keepdims=True)
    S_f = jnp.sum(f, axis=1, keepdims=True)
    u = S_g * f - f * g
    S_u = jnp.sum(u, axis=1, keepdims=True)
    d = (S_u - u) - g * (S_f - f)
    inv_d = jnp.where(d == 0.0, 0.0, 1.0 / d)
    r = h * inv_d
    w = g * r
    R_ = jnp.sum(r, axis=1, keepdims=True)
    Wt = jnp.sum(w, axis=1, keepdims=True)
    v = u * (R_ - r) - f * (Wt - w)
    S_v = jnp.sum(v, axis=1, keepdims=True)
    deg2 = 1.0 + (S_v - v) - r * (S_u - u) + w * (S_f - f)
    inv2 = jnp.where(deg2 == 0.0, 0.0, 1.0 / deg2)
    beta = inv2 * (1.0 - v + r * u - w * f)
    zero = jnp.zeros_like(f)
    return jnp.concatenate(
        [inv2, inv2 * r, inv2 * w, beta, v, u, f, zero], axis=0)


def _tdot(a, b):
    return jax.lax.dot_general(a, b, (((0,), (0,)), ((), ())),
                               preferred_element_type=jnp.float32)


NCHUNK = 2
CHUNK = N // NCHUNK


def _body(x_hbm, w1_ref, w2_ref, w3_ref, wgcn_hbm, linw_hbm, linb_ref,
          out_hbm, xv, gv, lv, ov, sx, sg, sl, so):
    cp_g = pltpu.make_async_copy(wgcn_hbm, gv, sg)
    cp_x = pltpu.make_async_copy(x_hbm, xv, sx)
    cp_l = pltpu.make_async_copy(linw_hbm, lv, sl)
    cp_g.start()
    cp_x.start()

    trows = [
        _coeff_rows(w1_ref[c:c + 1, :], w2_ref[c:c + 1, :], w3_ref[c:c + 1, :])
        for c in range(NUM_CHANNELS)
    ]
    eye8 = jnp.eye(8, dtype=jnp.float32)
    tcols = [_tdot(t, eye8) for t in trows]

    cp_g.wait()
    g_bf = gv[...].astype(jnp.bfloat16)
    cp_x.wait()
    cp_l.start()
    Xw = jnp.dot(xv[...].astype(jnp.bfloat16), g_bf,
                 preferred_element_type=jnp.float32)

    Ms = []
    for c in range(NUM_CHANNELS):
        R8 = jnp.dot(trows[c], Xw, preferred_element_type=jnp.float32)
        Ms.append(jnp.concatenate(
            [R8[4:5, :], -R8[5:6, :], R8[6:7, :],
             jnp.zeros((5, OUT_CH), dtype=jnp.float32)], axis=0))

    outs = []
    for c in range(NUM_CHANNELS):
        corr = jnp.dot(tcols[c], Ms[c], preferred_element_type=jnp.float32)
        beta = tcols[c][:, 3:4]
        outs.append(
            jnp.maximum(beta * Xw + corr, 0.0).astype(jnp.bfloat16))

    cp_l.wait()
    l_bf = lv[...].astype(jnp.bfloat16)
    cp_o = []
    for k in range(NCHUNK):
        sl_k = pl.ds(k * CHUNK, CHUNK)
        acc = None
        for c in range(NUM_CHANNELS):
            part = jnp.dot(outs[c][k * CHUNK:(k + 1) * CHUNK, :],
                           l_bf[c * OUT_CH:(c + 1) * OUT_CH, :],
                           preferred_element_type=jnp.float32)
            acc = part if acc is None else acc + part
        ov[sl_k, :] = jnp.maximum(acc + linb_ref[...], 0.0)
        cp = pltpu.make_async_copy(ov.at[sl_k, :], out_hbm.at[sl_k, :],
                                   so.at[k])
        cp.start()
        cp_o.append(cp)
    for cp in cp_o:
        cp.wait()


def kernel(x, W1, W2, W3, Wgcn, lin_w, lin_b):
    return pl.pallas_call(
        _body,
        in_specs=[
            pl.BlockSpec(memory_space=pltpu.HBM),
            pl.BlockSpec(memory_space=pltpu.VMEM),
            pl.BlockSpec(memory_space=pltpu.VMEM),
            pl.BlockSpec(memory_space=pltpu.VMEM),
            pl.BlockSpec(memory_space=pltpu.HBM),
            pl.BlockSpec(memory_space=pltpu.HBM),
            pl.BlockSpec(memory_space=pltpu.VMEM),
        ],
        out_specs=pl.BlockSpec(memory_space=pltpu.HBM),
        out_shape=jax.ShapeDtypeStruct((N, OUT_CH), jnp.float32),
        scratch_shapes=[
            pltpu.VMEM((N, IN_CH), jnp.float32),
            pltpu.VMEM((IN_CH, OUT_CH), jnp.float32),
            pltpu.VMEM((OUT_CH * NUM_CHANNELS, OUT_CH), jnp.float32),
            pltpu.VMEM((N, OUT_CH), jnp.float32),
            pltpu.SemaphoreType.DMA,
            pltpu.SemaphoreType.DMA,
            pltpu.SemaphoreType.DMA,
            pltpu.SemaphoreType.DMA((NCHUNK,)),
        ],
    )(x, W1, W2, W3, Wgcn, lin_w, lin_b.reshape(1, OUT_CH))
